# baseline (speedup 1.0000x reference)
import functools

import jax
import jax.numpy as jnp
from jax import lax
from jax.experimental import pallas as pl
from jax.experimental.pallas import tpu as pltpu
from jax.experimental.pallas import tpu_sc as plsc

NC, NS = 2, 16
NW = NC * NS
LANES = 16

B, L, D = 16384, 200, 8
BINS = 256
RPW = B // NW
BC = 128
NBC = RPW // BC
TT = 25
NTT = L // TT


def _vgather(src, idx):
  dnums = lax.GatherDimensionNumbers(
      offset_dims=(), collapsed_slice_dims=(0,), start_index_map=(0,))
  return lax.gather(src, idx[:, None], dnums, (1,),
                    mode=lax.GatherScatterMode.PROMISE_IN_BOUNDS)


def _sc_body(x_hbm, len_hbm, tbl_hbm, out_hbm, slen_hbm, rank_hbm,
             len_v, pos_v, idx_v, rank_v, slen_v, tbl_v,
             xb0, xb1, ob0, ob1, sxa, sxb, so0, so1, sem):
  wid = lax.axis_index("s") * NC + lax.axis_index("c")
  lane = lax.iota(jnp.int32, LANES)
  lane_tok = lane >> 3
  lane_d = lane & 7
  pbase = wid * RPW

  pltpu.sync_copy(len_hbm, len_v)
  pltpu.sync_copy(tbl_hbm, tbl_v)

  for t in range(BINS // LANES):
    pos_v[pl.ds(t * LANES, LANES)] = jnp.zeros((LANES,), jnp.int32)

  def hist_body(k2, _):
    for u in range(4):
      vals = len_v[pl.ds((k2 * 4 + u) * LANES, LANES)]
      occ, last = plsc.scan_count(vals)
      plsc.addupdate_scatter(pos_v, [vals], occ, mask=last)
    return 0

  lax.fori_loop(0, B // LANES // 4, hist_body, 0)

  carry = jnp.int32(0)
  for blk in reversed(range(BINS // LANES)):
    g = pos_v[pl.ds(blk * LANES, LANES)]
    tot = jnp.sum(g)
    incl = plsc.cumsum(g)
    pos_v[pl.ds(blk * LANES, LANES)] = carry + tot - incl
    carry = carry + tot

  def rank_body(k2, _):
    for u in range(2):
      k = k2 * 2 + u
      vals = len_v[pl.ds(k * LANES, LANES)]
      occ, last = plsc.scan_count(vals)
      base = plsc.load_gather(pos_v, [vals])
      rank = base + occ - 1
      plsc.store_scatter(pos_v, [vals], base + occ, mask=last)

      @pl.when((k >= wid * (RPW // LANES)) & (k < (wid + 1) * (RPW // LANES)))
      def _():
        rank_v[pl.ds((k - wid * (RPW // LANES)) * LANES, LANES)] = rank

      rloc = rank - pbase
      mine = (rloc >= 0) & (rloc < RPW)
      ivec = k * LANES + lane
      plsc.store_scatter(idx_v, [jnp.where(mine, rloc, 0)], ivec, mask=mine)
    return 0

  lax.fori_loop(0, B // LANES // 2, rank_body, 0)

  for t in range(RPW // LANES):
    iv = idx_v[pl.ds(t * LANES, LANES)]
    slen_v[pl.ds(t * LANES, LANES)] = plsc.load_gather(len_v, [iv])
  pltpu.sync_copy(rank_v, rank_hbm.at[pl.ds(wid * RPW, RPW)])
  pltpu.sync_copy(slen_v, slen_hbm.at[pl.ds(wid * RPW, RPW)])

  cols = [plsc.load_gather(tbl_v, [jnp.where(lane < D, lane * D + d, 0)])
          for d in range(D)]

  def expand_block(xb, ob, osem, c, s, first):
    def tt_body(tt, _):
      tvec = jnp.full((LANES,), s * TT + tt, jnp.int32)
      idvs = [plsc.load_gather(xb, [g * LANES + lane, tvec])
              for g in range(BC // LANES)]
      for g in range(BC // LANES):
        for d in range(D):
          ob[tt, 0, d, pl.ds(g * LANES, LANES)] = _vgather(cols[d], idvs[g])
      return 0

    dst = out_hbm.at[pl.ds(s * TT, TT), pl.ds(wid * NBC + c, 1), :, :]

    @pl.when(jnp.logical_not(first))
    def _():
      pltpu.make_async_copy(ob, dst, osem).wait()

    lax.fori_loop(0, TT, tt_body, 0)
    pltpu.async_copy(ob, dst, osem)

  def xgather(c, xb, xsem):
    return pltpu.async_copy(
        x_hbm.at[idx_v.at[pl.ds(c * BC, BC)]], xb, xsem)

  xgather(0, xb0, sxa)

  def cpair_body(cp, _):
    c0, c1 = 2 * cp, 2 * cp + 1
    pltpu.make_async_copy(x_hbm.at[idx_v.at[pl.ds(0, BC)]], xb0, sxa).wait()
    xgather(c1, xb1, sxb)

    def sp_body(sp, _):
      first = (cp == 0) & (sp == 0)
      expand_block(xb0, ob0, so0, c0, 2 * sp, first)
      expand_block(xb0, ob1, so1, c0, 2 * sp + 1, first)
      return 0

    lax.fori_loop(0, NTT // 2, sp_body, 0)
    pltpu.make_async_copy(x_hbm.at[idx_v.at[pl.ds(0, BC)]], xb1, sxb).wait()

    @pl.when(cp == 0)
    def _():
      xgather(2, xb0, sxa)

    def sp_body1(sp, _):
      expand_block(xb1, ob0, so0, c1, 2 * sp, False)
      expand_block(xb1, ob1, so1, c1, 2 * sp + 1, False)
      return 0

    lax.fori_loop(0, NTT // 2, sp_body1, 0)
    return 0

  lax.fori_loop(0, NBC // 2, cpair_body, 0)

  dst0 = out_hbm.at[pl.ds(0, TT), pl.ds(0, 1), :, :]
  pltpu.make_async_copy(ob0, dst0, so0).wait()
  pltpu.make_async_copy(ob1, dst0, so1).wait()


@jax.jit
def _sc_call(x, lengths, tbl_flat):
  mesh = plsc.VectorSubcoreMesh(core_axis_name="c", subcore_axis_name="s")
  f = pl.kernel(
      _sc_body, mesh=mesh,
      compiler_params=pltpu.CompilerParams(needs_layout_passes=False,
                                           use_tc_tiling_on_sc=False),
      out_type=(
          jax.ShapeDtypeStruct((L, B // BC, D, BC), jnp.float32),
          jax.ShapeDtypeStruct((B,), jnp.int32),
          jax.ShapeDtypeStruct((B,), jnp.int32),
      ),
      scratch_types=[
          pltpu.VMEM((B,), jnp.int32),
          pltpu.VMEM((BINS,), jnp.int32),
          pltpu.VMEM((RPW,), jnp.int32),
          pltpu.VMEM((RPW,), jnp.int32),
          pltpu.VMEM((RPW,), jnp.int32),
          pltpu.VMEM((D * D,), jnp.float32),
          pltpu.VMEM((BC, L), jnp.int32),
          pltpu.VMEM((BC, L), jnp.int32),
          pltpu.VMEM((TT, 1, D, BC), jnp.float32),
          pltpu.VMEM((TT, 1, D, BC), jnp.float32),
          pltpu.SemaphoreType.DMA,
          pltpu.SemaphoreType.DMA,
          pltpu.SemaphoreType.DMA,
          pltpu.SemaphoreType.DMA,
          pltpu.SemaphoreType.DMA,
      ],
  )
  return f(x, lengths, tbl_flat)


def kernel(x, lengths, table):
  emb4, slen, rank = _sc_call(
      x.astype(jnp.int32), lengths, table.reshape(D * D))
  return emb4.transpose(1, 3, 0, 2).reshape(B, L, D), slen, rank


# speedup vs baseline: 1.2672x; 1.2672x over previous
import functools

import jax
import jax.numpy as jnp
from jax import lax
from jax.experimental import pallas as pl
from jax.experimental.pallas import tpu as pltpu
from jax.experimental.pallas import tpu_sc as plsc

NC, NS = 2, 16
NW = NC * NS
LANES = 16

B, L, D = 16384, 200, 8
BINS = 256
RPW = B // NW
SLICE = B // NS
BC = 128
NBC = RPW // BC
TT = 25
NTT = L // TT


def _vgather(src, idx):
  dnums = lax.GatherDimensionNumbers(
      offset_dims=(), collapsed_slice_dims=(0,), start_index_map=(0,))
  return lax.gather(src, idx[:, None], dnums, (1,),
                    mode=lax.GatherScatterMode.PROMISE_IN_BOUNDS)


def _sc_body(x_hbm, len_hbm, tbl_hbm, out_hbm, slen_hbm, rank_hbm,
             len_v, pos_v, pre_v, hall_v, rank1k_v, ival_v,
             idx_v, slen_v, tbl_v,
             xb0, xb1, ob0, ob1, sh_hist, sh_idx,
             sxa, sxb, so0, so1, sem):
  cid = lax.axis_index("c")
  sid = lax.axis_index("s")
  wid = sid * NC + cid
  lane = lax.iota(jnp.int32, LANES)
  pbase = wid * RPW
  base_i = sid * SLICE

  pltpu.sync_copy(len_hbm, len_v)
  pltpu.sync_copy(tbl_hbm, tbl_v)

  for t in range(BINS // LANES):
    pos_v[pl.ds(t * LANES, LANES)] = jnp.zeros((LANES,), jnp.int32)

  def hist_body(k2, _):
    for u in range(4):
      vals = len_v[pl.ds(base_i + (k2 * 4 + u) * LANES, LANES)]
      occ, last = plsc.scan_count(vals)
      plsc.addupdate_scatter(pos_v, [vals], occ, mask=last)
    return 0

  lax.fori_loop(0, SLICE // LANES // 4, hist_body, 0)
  pltpu.sync_copy(pos_v, sh_hist.at[sid])
  plsc.subcore_barrier()
  pltpu.sync_copy(sh_hist, hall_v)

  for blk in range(BINS // LANES):
    tot = jnp.zeros((LANES,), jnp.int32)
    pre = jnp.zeros((LANES,), jnp.int32)
    for sp in range(NS):
      h = hall_v[sp, pl.ds(blk * LANES, LANES)]
      tot = tot + h
      pre = pre + jnp.where(sp < sid, h, 0)
    pos_v[pl.ds(blk * LANES, LANES)] = tot
    pre_v[pl.ds(blk * LANES, LANES)] = pre
  carry = jnp.int32(0)
  for blk in reversed(range(BINS // LANES)):
    g = pos_v[pl.ds(blk * LANES, LANES)]
    tot = jnp.sum(g)
    incl = plsc.cumsum(g)
    pos_v[pl.ds(blk * LANES, LANES)] = (
        carry + tot - incl + pre_v[pl.ds(blk * LANES, LANES)])
    carry = carry + tot

  def rank_body(k2, _):
    for u in range(2):
      k = k2 * 2 + u
      off = k * LANES
      vals = len_v[pl.ds(base_i + off, LANES)]
      occ, last = plsc.scan_count(vals)
      base = plsc.load_gather(pos_v, [vals])
      rank = base + occ - 1
      plsc.store_scatter(pos_v, [vals], base + occ, mask=last)
      rank1k_v[pl.ds(off, LANES)] = rank
      ival_v[pl.ds(off, LANES)] = base_i + off + lane
    return 0

  lax.fori_loop(0, SLICE // LANES // 2, rank_body, 0)
  pltpu.sync_copy(rank1k_v.at[pl.ds(cid * RPW, RPW)],
                  rank_hbm.at[pl.ds(base_i + cid * RPW, RPW)])
  pltpu.sync_copy(ival_v, sh_idx.at[rank1k_v])
  plsc.subcore_barrier()
  pltpu.sync_copy(sh_idx.at[pl.ds(pbase, RPW)], idx_v)

  for t in range(RPW // LANES):
    iv = idx_v[pl.ds(t * LANES, LANES)]
    slen_v[pl.ds(t * LANES, LANES)] = plsc.load_gather(len_v, [iv])
  pltpu.sync_copy(slen_v, slen_hbm.at[pl.ds(wid * RPW, RPW)])

  cols = [plsc.load_gather(tbl_v, [jnp.where(lane < D, lane * D + d, 0)])
          for d in range(D)]

  def expand_block(xb, ob, osem, c, s, first):
    def tt_body(tt, _):
      tvec = jnp.full((LANES,), s * TT + tt, jnp.int32)
      idvs = [plsc.load_gather(xb, [g * LANES + lane, tvec])
              for g in range(BC // LANES)]
      for g in range(BC // LANES):
        for d in range(D):
          ob[tt, 0, d, pl.ds(g * LANES, LANES)] = _vgather(cols[d], idvs[g])
      return 0

    dst = out_hbm.at[pl.ds(s * TT, TT), pl.ds(wid * NBC + c, 1), :, :]

    @pl.when(jnp.logical_not(first))
    def _():
      pltpu.make_async_copy(ob, dst, osem).wait()

    lax.fori_loop(0, TT, tt_body, 0)
    pltpu.async_copy(ob, dst, osem)

  def xgather(c, xb, xsem):
    return pltpu.async_copy(
        x_hbm.at[idx_v.at[pl.ds(c * BC, BC)]], xb, xsem)

  xgather(0, xb0, sxa)

  def cpair_body(cp, _):
    c0, c1 = 2 * cp, 2 * cp + 1
    pltpu.make_async_copy(x_hbm.at[idx_v.at[pl.ds(0, BC)]], xb0, sxa).wait()
    xgather(c1, xb1, sxb)

    def sp_body(sp, _):
      first = (cp == 0) & (sp == 0)
      expand_block(xb0, ob0, so0, c0, 2 * sp, first)
      expand_block(xb0, ob1, so1, c0, 2 * sp + 1, first)
      return 0

    lax.fori_loop(0, NTT // 2, sp_body, 0)
    pltpu.make_async_copy(x_hbm.at[idx_v.at[pl.ds(0, BC)]], xb1, sxb).wait()

    @pl.when(cp == 0)
    def _():
      xgather(2, xb0, sxa)

    def sp_body1(sp, _):
      expand_block(xb1, ob0, so0, c1, 2 * sp, False)
      expand_block(xb1, ob1, so1, c1, 2 * sp + 1, False)
      return 0

    lax.fori_loop(0, NTT // 2, sp_body1, 0)
    return 0

  lax.fori_loop(0, NBC // 2, cpair_body, 0)

  dst0 = out_hbm.at[pl.ds(0, TT), pl.ds(0, 1), :, :]
  pltpu.make_async_copy(ob0, dst0, so0).wait()
  pltpu.make_async_copy(ob1, dst0, so1).wait()


@jax.jit
def _sc_call(x, lengths, tbl_flat):
  mesh = plsc.VectorSubcoreMesh(core_axis_name="c", subcore_axis_name="s")
  f = pl.kernel(
      _sc_body, mesh=mesh,
      compiler_params=pltpu.CompilerParams(needs_layout_passes=False,
                                           use_tc_tiling_on_sc=False),
      out_type=(
          jax.ShapeDtypeStruct((L, B // BC, D, BC), jnp.float32),
          jax.ShapeDtypeStruct((B,), jnp.int32),
          jax.ShapeDtypeStruct((B,), jnp.int32),
      ),
      scratch_types=[
          pltpu.VMEM((B,), jnp.int32),
          pltpu.VMEM((BINS,), jnp.int32),
          pltpu.VMEM((BINS,), jnp.int32),
          pltpu.VMEM((NS, BINS), jnp.int32),
          pltpu.VMEM((SLICE,), jnp.int32),
          pltpu.VMEM((SLICE,), jnp.int32),
          pltpu.VMEM((RPW,), jnp.int32),
          pltpu.VMEM((RPW,), jnp.int32),
          pltpu.VMEM((D * D,), jnp.float32),
          pltpu.VMEM((BC, L), jnp.int32),
          pltpu.VMEM((BC, L), jnp.int32),
          pltpu.VMEM((TT, 1, D, BC), jnp.float32),
          pltpu.VMEM((TT, 1, D, BC), jnp.float32),
          pltpu.VMEM_SHARED((NS, BINS), jnp.int32),
          pltpu.VMEM_SHARED((B,), jnp.int32),
          pltpu.SemaphoreType.DMA,
          pltpu.SemaphoreType.DMA,
          pltpu.SemaphoreType.DMA,
          pltpu.SemaphoreType.DMA,
          pltpu.SemaphoreType.DMA,
      ],
  )
  return f(x, lengths, tbl_flat)


def kernel(x, lengths, table):
  emb4, slen, rank = _sc_call(
      x.astype(jnp.int32), lengths, table.reshape(D * D))
  return emb4.transpose(1, 3, 0, 2).reshape(B, L, D), slen, rank
